# baseline (speedup 1.0000x reference)
import functools

import jax
import jax.numpy as jnp
from jax import lax
from jax.experimental import pallas as pl
from jax.experimental.pallas import tpu as pltpu
from jax.experimental.pallas import tpu_sc as plsc

R, C = 64, 100000
BC = 2048
GRID = (C + BC - 1) // BC


def _rotl(x, r):
    return (x << jnp.uint32(r)) | (x >> jnp.uint32(32 - r))


def _threefry_bits(n):
    k0 = jnp.uint32(0)
    k1 = jnp.uint32(42)
    ks = [k0, k1, jnp.uint32(0x1BD11BDA) ^ k0 ^ k1]
    rot_even = (13, 15, 26, 6)
    rot_odd = (17, 29, 16, 24)
    t = n + k1
    x0 = t
    x1 = _rotl(t, 13) ^ t
    for r in rot_even[1:]:
        x0 = x0 + x1
        x1 = _rotl(x1, r)
        x1 = x1 ^ x0
    x0 = x0 + ks[1]
    x1 = x1 + ks[2] + jnp.uint32(1)
    for i in range(1, 5):
        for r in rot_even if i % 2 == 0 else rot_odd:
            x0 = x0 + x1
            x1 = _rotl(x1, r)
            x1 = x1 ^ x0
        x0 = x0 + ks[(i + 1) % 3]
        x1 = x1 + ks[(i + 2) % 3] + jnp.uint32(i + 1)
    return x0 ^ x1


def _gumbel(rows, cols):
    n = (rows * C + cols).astype(jnp.uint32)
    bits = _threefry_bits(n)
    fb = (bits >> jnp.uint32(9)) | jnp.uint32(0x3F800000)
    u = jax.lax.bitcast_convert_type(fb, jnp.float32) - jnp.float32(1.0)
    eps = jnp.float32(1e-10)
    return -jnp.log(-jnp.log(u + eps) + eps)


def _argmax_body(x_ref, o_ref, maxv_ref, argf_ref):
    step = pl.program_id(0)
    shape = (R, BC)
    cols = jax.lax.broadcasted_iota(jnp.int32, shape, 1) + step * BC
    rows = jax.lax.broadcasted_iota(jnp.int32, shape, 0)
    y = x_ref[...] + _gumbel(rows, cols)
    y = jnp.where(cols < C, y, -jnp.inf)
    m = jnp.max(y, axis=1, keepdims=True)
    cand = jnp.where(y == m, cols, jnp.int32(2**31 - 1))
    a = jnp.min(cand, axis=1, keepdims=True)
    o_ref[...] = jnp.zeros(shape, jnp.float32)

    @pl.when(step == 0)
    def _():
        maxv_ref[...] = m
        argf_ref[...] = a

    @pl.when(step > 0)
    def _():
        upd = m > maxv_ref[...]
        maxv_ref[...] = jnp.where(upd, m, maxv_ref[...])
        argf_ref[...] = jnp.where(upd, a, argf_ref[...])

    @pl.when(step == GRID - 1)
    def _():
        riota = jax.lax.broadcasted_iota(jnp.int32, (R, 1), 0)
        argf_ref[...] = riota * C + argf_ref[...]


_sc_mesh = plsc.VectorSubcoreMesh(core_axis_name="c", subcore_axis_name="s")


@functools.partial(
    pl.kernel,
    mesh=_sc_mesh,
    scratch_types=[
        pltpu.VMEM((R,), jnp.int32),
        pltpu.VMEM((R,), jnp.float32),
        pltpu.SemaphoreType.DMA,
    ],
)
def _sc_scatter_ones(idx_hbm, out_hbm, idx_v, ones_v, sem):
    @pl.when((lax.axis_index("c") == 0) & (lax.axis_index("s") == 0))
    def _():
        pltpu.sync_copy(idx_hbm, idx_v)
        for i in range(R // 16):
            ones_v[pl.ds(i * 16, 16)] = jnp.ones((16,), jnp.float32)
        pltpu.async_copy(ones_v, out_hbm.at[idx_v], sem).wait()


@jax.jit
def kernel(logits):
    zeros_out, _, argf = pl.pallas_call(
        _argmax_body,
        grid=(GRID,),
        in_specs=[pl.BlockSpec((R, BC), lambda i: (0, i))],
        out_specs=[
            pl.BlockSpec((R, BC), lambda i: (0, i)),
            pl.BlockSpec((R, 1), lambda i: (0, 0)),
            pl.BlockSpec((R, 1), lambda i: (0, 0)),
        ],
        out_shape=[
            jax.ShapeDtypeStruct((R, C), jnp.float32),
            jax.ShapeDtypeStruct((R, 1), jnp.float32),
            jax.ShapeDtypeStruct((R, 1), jnp.int32),
        ],
        compiler_params=pltpu.CompilerParams(
            dimension_semantics=("arbitrary",)),
    )(logits)
    out_ref = jax.new_ref(zeros_out.reshape(R * C))
    _sc_scatter_ones(argf.reshape(R), out_ref)
    return out_ref[...].reshape(R, C)


# speedup vs baseline: 1.1068x; 1.1068x over previous
import functools

import jax
import jax.numpy as jnp
from jax import lax
from jax.experimental import pallas as pl
from jax.experimental.pallas import tpu as pltpu
from jax.experimental.pallas import tpu_sc as plsc

R, C = 64, 100000
BC = 2048
GRID = (C + BC - 1) // BC


def _rotl(x, r):
    return (x << jnp.uint32(r)) | (x >> jnp.uint32(32 - r))


def _threefry_bits(n):
    k0 = jnp.uint32(0)
    k1 = jnp.uint32(42)
    ks = [k0, k1, jnp.uint32(0x1BD11BDA) ^ k0 ^ k1]
    rot_even = (13, 15, 26, 6)
    rot_odd = (17, 29, 16, 24)
    t = n + k1
    x0 = t
    x1 = _rotl(t, 13) ^ t
    for r in rot_even[1:]:
        x0 = x0 + x1
        x1 = _rotl(x1, r)
        x1 = x1 ^ x0
    x0 = x0 + ks[1]
    x1 = x1 + ks[2] + jnp.uint32(1)
    for i in range(1, 5):
        for r in rot_even if i % 2 == 0 else rot_odd:
            x0 = x0 + x1
            x1 = _rotl(x1, r)
            x1 = x1 ^ x0
        x0 = x0 + ks[(i + 1) % 3]
        x1 = x1 + ks[(i + 2) % 3] + jnp.uint32(i + 1)
    return x0 ^ x1


def _gumbel(rows, cols):
    n = (rows * C + cols).astype(jnp.uint32)
    bits = _threefry_bits(n)
    fb = (bits >> jnp.uint32(9)) | jnp.uint32(0x3F800000)
    u = jax.lax.bitcast_convert_type(fb, jnp.float32) - jnp.float32(1.0)
    eps = jnp.float32(1e-10)
    return -jnp.log(-jnp.log(u + eps) + eps)


def _argmax_body(x_ref, o_ref, maxv_ref, argf_ref):
    step = pl.program_id(0)
    shape = (R, BC)
    cols = jax.lax.broadcasted_iota(jnp.int32, shape, 1) + step * BC
    rows = jax.lax.broadcasted_iota(jnp.int32, shape, 0)
    y = x_ref[...] + _gumbel(rows, cols)
    y = jnp.where(cols < C, y, -jnp.inf)
    m = jnp.max(y, axis=1, keepdims=True)
    cand = jnp.where(y == m, cols, jnp.int32(2**31 - 1))
    a = jnp.min(cand, axis=1, keepdims=True)
    o_ref[...] = jnp.zeros(shape, jnp.float32)

    @pl.when(step == 0)
    def _():
        maxv_ref[...] = m
        argf_ref[...] = a

    @pl.when(step > 0)
    def _():
        upd = m > maxv_ref[...]
        maxv_ref[...] = jnp.where(upd, m, maxv_ref[...])
        argf_ref[...] = jnp.where(upd, a, argf_ref[...])


_sc_mesh = plsc.VectorSubcoreMesh(core_axis_name="c", subcore_axis_name="s")


@functools.partial(
    pl.kernel,
    mesh=_sc_mesh,
    scratch_types=[
        pltpu.VMEM((R,), jnp.int32),
        pltpu.VMEM((R,), jnp.float32),
        pltpu.SemaphoreType.DMA,
    ],
)
def _sc_scatter_ones(idx_hbm, out_hbm, idx_v, ones_v, sem):
    @pl.when((lax.axis_index("c") == 0) & (lax.axis_index("s") == 0))
    def _():
        pltpu.sync_copy(idx_hbm, idx_v)
        for i in range(R // 16):
            ones_v[pl.ds(i * 16, 16)] = jnp.ones((16,), jnp.float32)
        pltpu.async_copy(ones_v, out_hbm.at[idx_v], sem).wait()


@jax.jit
def kernel(logits):
    zeros_out, _, argf = pl.pallas_call(
        _argmax_body,
        grid=(GRID,),
        in_specs=[pl.BlockSpec((R, BC), lambda i: (0, i))],
        out_specs=[
            pl.BlockSpec((R, BC), lambda i: (0, i)),
            pl.BlockSpec((R, 1), lambda i: (0, 0)),
            pl.BlockSpec((R, 1), lambda i: (0, 0)),
        ],
        out_shape=[
            jax.ShapeDtypeStruct((R, C), jnp.float32),
            jax.ShapeDtypeStruct((R, 1), jnp.float32),
            jax.ShapeDtypeStruct((R, 1), jnp.int32),
        ],
        compiler_params=pltpu.CompilerParams(
            dimension_semantics=("arbitrary",)),
    )(logits)
    out = _patch_ones(argf.reshape(R), zeros_out)
    return out


def _patch_body(argc_ref, z_ref, o_ref):
    r = pl.program_id(0)
    col = argc_ref[r]
    base = (col // 128) * 128
    lanes = jax.lax.broadcasted_iota(jnp.int32, (1, 1, 128), 2) + base
    o_ref[...] = (lanes == col).astype(jnp.float32)


def _patch_ones(argc, zeros_out):
    z3 = zeros_out.reshape(R, 1, C)
    out = pl.pallas_call(
        _patch_body,
        grid_spec=pltpu.PrefetchScalarGridSpec(
            num_scalar_prefetch=1,
            grid=(R,),
            in_specs=[
                pl.BlockSpec((1, 1, 128), lambda i, pref: (i, 0, pref[i] // 128)),
            ],
            out_specs=pl.BlockSpec(
                (1, 1, 128), lambda i, pref: (i, 0, pref[i] // 128)),
        ),
        out_shape=jax.ShapeDtypeStruct((R, 1, C), jnp.float32),
        input_output_aliases={1: 0},
        compiler_params=pltpu.CompilerParams(
            dimension_semantics=("arbitrary",)),
    )(argc, z3)
    return out.reshape(R, C)


# speedup vs baseline: 1.6436x; 1.4850x over previous
import functools

import jax
import jax.numpy as jnp
from jax import lax
from jax.experimental import pallas as pl
from jax.experimental.pallas import tpu as pltpu
from jax.experimental.pallas import tpu_sc as plsc

R, C = 64, 100000
BC = 2048
GRID = (C + BC - 1) // BC


def _rotl(x, r):
    return (x << jnp.uint32(r)) | (x >> jnp.uint32(32 - r))


def _threefry_bits(n):
    k0 = jnp.uint32(0)
    k1 = jnp.uint32(42)
    ks = [k0, k1, jnp.uint32(0x1BD11BDA) ^ k0 ^ k1]
    rot_even = (13, 15, 26, 6)
    rot_odd = (17, 29, 16, 24)
    t = n + k1
    x0 = t
    x1 = _rotl(t, 13) ^ t
    for r in rot_even[1:]:
        x0 = x0 + x1
        x1 = _rotl(x1, r)
        x1 = x1 ^ x0
    x0 = x0 + ks[1]
    x1 = x1 + ks[2] + jnp.uint32(1)
    for i in range(1, 5):
        for r in rot_even if i % 2 == 0 else rot_odd:
            x0 = x0 + x1
            x1 = _rotl(x1, r)
            x1 = x1 ^ x0
        x0 = x0 + ks[(i + 1) % 3]
        x1 = x1 + ks[(i + 2) % 3] + jnp.uint32(i + 1)
    return x0 ^ x1


def _gumbel(rows, cols):
    n = (rows * C + cols).astype(jnp.uint32)
    bits = _threefry_bits(n)
    fb = (bits >> jnp.uint32(9)) | jnp.uint32(0x3F800000)
    u = jax.lax.bitcast_convert_type(fb, jnp.float32) - jnp.float32(1.0)
    eps = jnp.float32(1e-10)
    return -jnp.log(-jnp.log(u + eps) + eps)


def _argmax_body(x_ref, o_ref, maxv_ref, argf_ref):
    step = pl.program_id(0)
    shape = (R, BC)
    cols = jax.lax.broadcasted_iota(jnp.int32, shape, 1) + step * BC
    rows = jax.lax.broadcasted_iota(jnp.int32, shape, 0)
    y = x_ref[...] + _gumbel(rows, cols)
    y = jnp.where(cols < C, y, -jnp.inf)
    m = jnp.max(y, axis=1, keepdims=True)
    cand = jnp.where(y == m, cols, jnp.int32(2**31 - 1))
    a = jnp.min(cand, axis=1, keepdims=True)

    @pl.when(step == 0)
    def _():
        maxv_ref[...] = m
        argf_ref[...] = a

    @pl.when(step > 0)
    def _():
        upd = m > maxv_ref[...]
        maxv_ref[...] = jnp.where(upd, m, maxv_ref[...])
        argf_ref[...] = jnp.where(upd, a, argf_ref[...])

    @pl.when(step < GRID - 1)
    def _():
        o_ref[...] = jnp.zeros(shape, jnp.float32)

    @pl.when(step == GRID - 1)
    def _():
        o_ref[...] = (cols == argf_ref[...]).astype(jnp.float32)


_sc_mesh = plsc.VectorSubcoreMesh(core_axis_name="c", subcore_axis_name="s")


@functools.partial(
    pl.kernel,
    mesh=_sc_mesh,
    scratch_types=[
        pltpu.VMEM((R,), jnp.int32),
        pltpu.VMEM((R,), jnp.float32),
        pltpu.SemaphoreType.DMA,
    ],
)
def _sc_scatter_ones(idx_hbm, out_hbm, idx_v, ones_v, sem):
    @pl.when((lax.axis_index("c") == 0) & (lax.axis_index("s") == 0))
    def _():
        pltpu.sync_copy(idx_hbm, idx_v)
        for i in range(R // 16):
            ones_v[pl.ds(i * 16, 16)] = jnp.ones((16,), jnp.float32)
        pltpu.async_copy(ones_v, out_hbm.at[idx_v], sem).wait()


@jax.jit
def kernel(logits):
    zeros_out, _, argf = pl.pallas_call(
        _argmax_body,
        grid=(GRID,),
        in_specs=[pl.BlockSpec((R, BC), lambda i: (0, i))],
        out_specs=[
            pl.BlockSpec((R, BC), lambda i: (0, i)),
            pl.BlockSpec((R, 1), lambda i: (0, 0)),
            pl.BlockSpec((R, 1), lambda i: (0, 0)),
        ],
        out_shape=[
            jax.ShapeDtypeStruct((R, C), jnp.float32),
            jax.ShapeDtypeStruct((R, 1), jnp.float32),
            jax.ShapeDtypeStruct((R, 1), jnp.int32),
        ],
        compiler_params=pltpu.CompilerParams(
            dimension_semantics=("arbitrary",)),
    )(logits)
    return _patch_ones(argf, zeros_out)


_LAST_BASE = (GRID - 1) * BC
_MAX_WIN = _LAST_BASE - 128


def _patch_body(argc_v_ref, argc_s_ref, z_ref, o_ref, pat_ref, sem):
    copies = []
    for r in range(R):
        g = r // 8
        base = jnp.minimum((argc_s_ref[r, 0] // 128) * 128, _MAX_WIN)
        argc_g = argc_v_ref[pl.ds(8 * g, 8), :]
        lanes = jax.lax.broadcasted_iota(jnp.int32, (8, 128), 1) + base
        pat_ref[r] = (argc_g == lanes).astype(jnp.float32)
        cp = pltpu.make_async_copy(
            pat_ref.at[r],
            o_ref.at[pl.ds(8 * g, 8), pl.ds(base, 128)],
            sem)
        cp.start()
        copies.append(cp)
    for cp in copies:
        cp.wait()


def _patch_ones(argc, zeros_out):
    return pl.pallas_call(
        _patch_body,
        in_specs=[
            pl.BlockSpec(memory_space=pltpu.VMEM),
            pl.BlockSpec(memory_space=pltpu.SMEM),
            pl.BlockSpec(memory_space=pl.ANY),
        ],
        out_specs=pl.BlockSpec(memory_space=pl.ANY),
        out_shape=jax.ShapeDtypeStruct((R, C), jnp.float32),
        scratch_shapes=[
            pltpu.VMEM((R, 8, 128), jnp.float32),
            pltpu.SemaphoreType.DMA,
        ],
        input_output_aliases={2: 0},
    )(argc, argc, zeros_out)
